# baseline (speedup 1.0000x reference)
import jax
import jax.numpy as jnp
from jax import lax
from jax.experimental import pallas as pl
from jax.experimental.pallas import tpu as pltpu
from jax.experimental.pallas import tpu_sc as plsc

D = 128
LANES = 16
DCHUNKS = D // LANES
NSUB = 4


def _embed_body(x_hbm, seg_hbm, tok_hbm, pos_hbm, segtab_hbm, out_hbm,
                idx_v, seg_v, rows_v, pos_v, segtab_v, pos_sh,
                tok_sems, pos_sems, out_sem):
    nc = 2
    c = lax.axis_index("c")
    s = lax.axis_index("s")
    wid = s * nc + c
    chunk = rows_v.shape[0]
    sub = chunk // NSUB
    seq = pos_hbm.shape[0]
    base = wid * chunk
    bb = lax.div(base, seq)
    off = lax.rem(base, seq)

    pltpu.sync_copy(seg_hbm.at[bb, pl.ds(off, chunk)], seg_v)
    pltpu.sync_copy(segtab_hbm, segtab_v)

    copies = []
    for k in range(NSUB):
        sl = pl.ds(k * sub, sub)
        pltpu.sync_copy(x_hbm.at[bb, pl.ds(off + k * sub, sub)], idx_v.at[k])
        tok_cp = pltpu.async_copy(tok_hbm.at[idx_v.at[k]], rows_v.at[sl],
                                  tok_sems[k])
        copies.append(tok_cp)

    nslots = pos_sh.shape[0] // chunk
    @pl.when(s < nslots)
    def _load_pos():
        g = s
        pltpu.sync_copy(pos_hbm.at[pl.ds((nc * g + c) * chunk, chunk)],
                        pos_sh.at[pl.ds(g * chunk, chunk)])
    plsc.subcore_barrier()

    slot = lax.div(lax.rem(wid, 2 * nslots) - c, nc)
    pos_copies = []
    for k in range(NSUB):
        sl = pl.ds(k * sub, sub)
        pos_cp = pltpu.async_copy(pos_sh.at[pl.ds(slot * chunk + k * sub, sub)],
                                  pos_v.at[sl], pos_sems[k])
        pos_copies.append(pos_cp)

    seg0 = [segtab_v[0, pl.ds(j * LANES, LANES)] for j in range(DCHUNKS)]
    dif = [segtab_v[1, pl.ds(j * LANES, LANES)] - seg0[j] for j in range(DCHUNKS)]

    def grp_body(g, carry):
        sv = seg_v[pl.ds(g * LANES, LANES)].astype(jnp.float32)

        def tok_body(t, c2):
            i = g * LANES + t
            sf = jnp.take(sv, jnp.full((LANES,), t, jnp.int32),
                          mode="fill")
            toks = [rows_v[i, pl.ds(j * LANES, LANES)] for j in range(DCHUNKS)]
            poss = [pos_v[i, pl.ds(j * LANES, LANES)] for j in range(DCHUNKS)]
            for j in range(DCHUNKS):
                rows_v[i, pl.ds(j * LANES, LANES)] = (
                    toks[j] + poss[j] + (seg0[j] + sf * dif[j]))
            return c2

        return lax.fori_loop(0, LANES, tok_body, carry)

    out_cps = []
    gps = sub // LANES
    for k in range(NSUB):
        copies[k].wait()
        pos_copies[k].wait()
        lax.fori_loop(k * gps, (k + 1) * gps, grp_body, 0)
        sl = pl.ds(k * sub, sub)
        out_cps.append(pltpu.async_copy(
            rows_v.at[sl],
            out_hbm.at[bb, pl.ds(off + k * sub, sub)],
            out_sem))
    for cp in out_cps:
        cp.wait()


def kernel(x, seg, tok_embed, pos_embed, seg_embed):
    batch, seq = x.shape
    n = batch * seq
    nw = 32
    chunk = n // nw
    sub = chunk // NSUB

    mesh = plsc.VectorSubcoreMesh(core_axis_name="c", subcore_axis_name="s")
    out = pl.kernel(
        _embed_body,
        out_type=jax.ShapeDtypeStruct((batch, seq, D), jnp.float32),
        mesh=mesh,
        scratch_types=[
            pltpu.VMEM((NSUB, sub), jnp.int32),
            pltpu.VMEM((chunk,), jnp.int32),
            pltpu.VMEM((chunk, D), jnp.float32),
            pltpu.VMEM((chunk, D), jnp.float32),
            pltpu.VMEM((2, D), jnp.float32),
            pltpu.VMEM_SHARED((seq // 2, D), jnp.float32),
            [pltpu.SemaphoreType.DMA] * NSUB,
            [pltpu.SemaphoreType.DMA] * NSUB,
            pltpu.SemaphoreType.DMA,
        ],
    )(x.astype(jnp.int32), seg.astype(jnp.int32), tok_embed, pos_embed,
      seg_embed)
    return out


# speedup vs baseline: 1.0438x; 1.0438x over previous
import jax
import jax.numpy as jnp
from jax import lax
from jax.experimental import pallas as pl
from jax.experimental.pallas import tpu as pltpu
from jax.experimental.pallas import tpu_sc as plsc

D = 128
LANES = 16
DCHUNKS = D // LANES
NW = 32
PWIN = 64


def _embed_body(x_hbm, seg_hbm, tok_hbm, pos_hbm, segtab_hbm, out_hbm,
                idx_v, seg_v, rows_v, pos_v, segtab_v,
                tok_sems, seg_sems, pos_sem, out_sem):
    nc = 2
    c = lax.axis_index("c")
    s = lax.axis_index("s")
    wid = s * nc + c
    batch = x_hbm.shape[0]
    pbase = wid * PWIN

    tok_cps = []
    seg_cps = []
    for b in range(batch):
        pltpu.sync_copy(x_hbm.at[b, pl.ds(pbase, PWIN)], idx_v.at[b])
        tok_cps.append(pltpu.async_copy(
            tok_hbm.at[idx_v.at[b]], rows_v.at[pl.ds(b * PWIN, PWIN)],
            tok_sems[b]))
        seg_cps.append(pltpu.async_copy(
            seg_hbm.at[b, pl.ds(pbase, PWIN)],
            seg_v.at[pl.ds(b * PWIN, PWIN)], seg_sems[b]))
    pos_cp = pltpu.async_copy(pos_hbm.at[pl.ds(pbase, PWIN)], pos_v, pos_sem)
    pltpu.sync_copy(segtab_hbm, segtab_v)

    seg0 = [segtab_v[0, pl.ds(j * LANES, LANES)] for j in range(DCHUNKS)]
    dif = [segtab_v[1, pl.ds(j * LANES, LANES)] - seg0[j] for j in range(DCHUNKS)]

    def make_grp_body(b):
        def grp_body(g, carry):
            sv = seg_v[pl.ds(b * PWIN + g * LANES, LANES)].astype(jnp.float32)

            def tok_body(t, c2):
                i = b * PWIN + g * LANES + t
                ip = g * LANES + t
                sf = jnp.take(sv, jnp.full((LANES,), t, jnp.int32),
                              mode="fill")
                toks = [rows_v[i, pl.ds(j * LANES, LANES)]
                        for j in range(DCHUNKS)]
                poss = [pos_v[ip, pl.ds(j * LANES, LANES)]
                        for j in range(DCHUNKS)]
                for j in range(DCHUNKS):
                    rows_v[i, pl.ds(j * LANES, LANES)] = (
                        toks[j] + poss[j] + (seg0[j] + sf * dif[j]))
                return c2

            return lax.fori_loop(0, LANES, tok_body, carry)
        return grp_body

    pos_cp.wait()
    out_cps = []
    for b in range(batch):
        tok_cps[b].wait()
        seg_cps[b].wait()
        lax.fori_loop(0, PWIN // LANES, make_grp_body(b), 0)
        out_cps.append(pltpu.async_copy(
            rows_v.at[pl.ds(b * PWIN, PWIN)],
            out_hbm.at[b, pl.ds(pbase, PWIN)], out_sem))
    for cp in out_cps:
        cp.wait()


def kernel(x, seg, tok_embed, pos_embed, seg_embed):
    batch, seq = x.shape
    mesh = plsc.VectorSubcoreMesh(core_axis_name="c", subcore_axis_name="s")
    out = pl.kernel(
        _embed_body,
        out_type=jax.ShapeDtypeStruct((batch, seq, D), jnp.float32),
        mesh=mesh,
        scratch_types=[
            pltpu.VMEM((batch, PWIN), jnp.int32),
            pltpu.VMEM((batch * PWIN,), jnp.int32),
            pltpu.VMEM((batch * PWIN, D), jnp.float32),
            pltpu.VMEM((PWIN, D), jnp.float32),
            pltpu.VMEM((2, D), jnp.float32),
            [pltpu.SemaphoreType.DMA] * batch,
            [pltpu.SemaphoreType.DMA] * batch,
            pltpu.SemaphoreType.DMA,
            pltpu.SemaphoreType.DMA,
        ],
    )(x.astype(jnp.int32), seg.astype(jnp.int32), tok_embed, pos_embed,
      seg_embed)
    return out


# speedup vs baseline: 1.0535x; 1.0093x over previous
import jax
import jax.numpy as jnp
from jax import lax
from jax.experimental import pallas as pl
from jax.experimental.pallas import tpu as pltpu
from jax.experimental.pallas import tpu_sc as plsc

D = 128
LANES = 16
DCHUNKS = D // LANES
NW = 32
PWIN = 64


def _embed_body(x_hbm, seg_hbm, tok_hbm, pos_hbm, segtab_hbm, out_hbm,
                idx_v, seg_v, rows_v, pos_v, segtab_v,
                tok_sems, seg_sems, pos_sem, out_sem):
    nc = 2
    c = lax.axis_index("c")
    s = lax.axis_index("s")
    wid = s * nc + c
    batch = x_hbm.shape[0]
    pbase = wid * PWIN

    idx_cps = [pltpu.async_copy(x_hbm.at[b, pl.ds(pbase, PWIN)], idx_v.at[b],
                                tok_sems[b]) for b in range(batch)]
    pos_cp = pltpu.async_copy(pos_hbm.at[pl.ds(pbase, PWIN)], pos_v, pos_sem)
    seg_cps = [pltpu.async_copy(seg_hbm.at[b, pl.ds(pbase, PWIN)],
                                seg_v.at[b], seg_sems[b]) for b in range(batch)]
    pltpu.sync_copy(segtab_hbm, segtab_v)
    for cp in idx_cps:
        cp.wait()
    tok_cps = []
    for b in range(batch):
        tok_cps.append(pltpu.async_copy(
            tok_hbm.at[idx_v.at[b]], rows_v.at[pl.ds(b * PWIN, PWIN)],
            tok_sems[b]))
    for cp in seg_cps:
        cp.wait()

    seg0 = [segtab_v[0, pl.ds(j * LANES, LANES)] for j in range(DCHUNKS)]
    dif = [segtab_v[1, pl.ds(j * LANES, LANES)] - seg0[j] for j in range(DCHUNKS)]

    def make_grp_body(b):
        def grp_body(g, carry):
            sv = seg_v[b, pl.ds(g * LANES, LANES)].astype(jnp.float32)

            def tok_body(t, c2):
                i = b * PWIN + g * LANES + t
                ip = g * LANES + t
                sf = jnp.take(sv, jnp.full((LANES,), t, jnp.int32),
                              mode="fill")
                toks = [rows_v[i, pl.ds(j * LANES, LANES)]
                        for j in range(DCHUNKS)]
                poss = [pos_v[ip, pl.ds(j * LANES, LANES)]
                        for j in range(DCHUNKS)]
                for j in range(DCHUNKS):
                    rows_v[i, pl.ds(j * LANES, LANES)] = (
                        toks[j] + poss[j] + (seg0[j] + sf * dif[j]))
                return c2

            return lax.fori_loop(0, LANES, tok_body, carry)
        return grp_body

    pos_cp.wait()
    out_cps = []
    for b in range(batch):
        tok_cps[b].wait()
        lax.fori_loop(0, PWIN // LANES, make_grp_body(b), 0)
        out_cps.append(pltpu.async_copy(
            rows_v.at[pl.ds(b * PWIN, PWIN)],
            out_hbm.at[b, pl.ds(pbase, PWIN)], out_sem))
    for cp in out_cps:
        cp.wait()


def kernel(x, seg, tok_embed, pos_embed, seg_embed):
    batch, seq = x.shape
    mesh = plsc.VectorSubcoreMesh(core_axis_name="c", subcore_axis_name="s")
    out = pl.kernel(
        _embed_body,
        out_type=jax.ShapeDtypeStruct((batch, seq, D), jnp.float32),
        mesh=mesh,
        scratch_types=[
            pltpu.VMEM((batch, PWIN), jnp.int32),
            pltpu.VMEM((batch, PWIN), jnp.int32),
            pltpu.VMEM((batch * PWIN, D), jnp.float32),
            pltpu.VMEM((PWIN, D), jnp.float32),
            pltpu.VMEM((2, D), jnp.float32),
            [pltpu.SemaphoreType.DMA] * batch,
            [pltpu.SemaphoreType.DMA] * batch,
            pltpu.SemaphoreType.DMA,
            pltpu.SemaphoreType.DMA,
        ],
    )(x.astype(jnp.int32), seg.astype(jnp.int32), tok_embed, pos_embed,
      seg_embed)
    return out
